# baseline (speedup 1.0000x reference)
import functools

import jax
import jax.numpy as jnp
from jax import lax
from jax.experimental import pallas as pl
from jax.experimental.pallas import tpu as pltpu
from jax.experimental.pallas import tpu_sc as plsc

CTX = 200
EMBED = 64
VOCAB = 1000000

_RING = 7
_NW = 32
_SLOTS = (CTX + _NW - 1) // _NW


def _sc_gather_sum_body(idx_hbm, tabt_hbm, out_hbm, idx_vm, blks, acc_v, *sems):
    c = lax.axis_index("c")
    t = lax.axis_index("s")
    wid = c * 16 + t

    pltpu.sync_copy(idx_hbm, idx_vm.at[pl.ds(0, CTX)])

    def get_v(j):
        chunk = idx_vm[pl.ds(32 * j + 16 * c, 16)]
        sel = lax.iota(jnp.int32, 16) == t
        return jnp.sum(jnp.where(sel, chunk, 0))

    def col_base(v):
        return pl.multiple_of(v - (v & 127), 128)

    def issue(j, b):
        pltpu.make_async_copy(
            tabt_hbm.at[:, pl.ds(col_base(get_v(j)), 128)], blks.at[b], sems[b]
        ).start()

    def active(j):
        return (wid + 32 * j < CTX) if 32 * j + 31 >= CTX else None

    def when_active(j, fn):
        a = active(j)
        if a is None:
            fn()
        else:
            pl.when(a)(fn)

    for b in range(min(_RING, _SLOTS)):
        when_active(b, lambda b=b: issue(b, b))

    acc = [jnp.zeros((16,), jnp.float32) for _ in range(4)]
    for j in range(_SLOTS):
        b = j % _RING
        when_active(j, lambda b=b: pltpu.make_async_copy(
            tabt_hbm.at[:, pl.ds(0, 128)], blks.at[b], sems[b]).wait())
        v = get_v(j)
        o = v - col_base(v)
        cols = jnp.full((16,), o, jnp.int32)
        a = active(j)
        for k in range(4):
            rows = lax.iota(jnp.int32, 16) + 16 * k
            g = plsc.load_gather(blks.at[b], [rows, cols])
            acc[k] = acc[k] + g if a is None else acc[k] + jnp.where(a, g, 0.0)
        if j + _RING < _SLOTS:
            when_active(j + _RING, lambda j=j, b=b: issue(j + _RING, b))

    for k in range(4):
        acc_v[k, :] = acc[k]
    pltpu.sync_copy(acc_v, out_hbm.at[wid])


@jax.jit
def _sc_gather_sum(inputs, tab_t):
    mesh = plsc.VectorSubcoreMesh(core_axis_name="c", subcore_axis_name="s")
    return pl.kernel(
        _sc_gather_sum_body,
        out_type=jax.ShapeDtypeStruct((_NW, 4, 16), jnp.float32),
        mesh=mesh,
        scratch_types=[
            pltpu.VMEM((32 * _SLOTS, ), jnp.int32),
            pltpu.VMEM((_RING, EMBED, 128), jnp.float32),
            pltpu.VMEM((4, 16), jnp.float32),
        ] + [pltpu.SemaphoreType.DMA] * _RING,
        compiler_params=pltpu.CompilerParams(needs_layout_passes=False),
    )(inputs, tab_t)


_BLKV = 35840


def _tc_matvec_body(p_ref, wt_ref, b_ref, o_ref):
    e = jnp.sum(p_ref[:], axis=0, keepdims=True)
    e8 = jnp.broadcast_to(e, (8, EMBED))
    acc = lax.dot_general(
        e8, wt_ref[:], (((1,), (0,)), ((), ())),
        preferred_element_type=jnp.float32,
    )
    o_ref[:] = acc[0] + b_ref[:]


@jax.jit
def _tc_matvec(partials, W_t, b):
    nblk = pl.cdiv(VOCAB, _BLKV)
    return pl.pallas_call(
        _tc_matvec_body,
        grid=(nblk,),
        in_specs=[
            pl.BlockSpec((32, EMBED), lambda i: (0, 0)),
            pl.BlockSpec((EMBED, _BLKV), lambda i: (0, i)),
            pl.BlockSpec((_BLKV,), lambda i: (i,)),
        ],
        out_specs=pl.BlockSpec((_BLKV,), lambda i: (i,)),
        out_shape=jax.ShapeDtypeStruct((VOCAB,), jnp.float32),
    )(partials, W_t, b)


def kernel(inputs, emb_table, W, b):
    partials = _sc_gather_sum(inputs, emb_table.T)
    return _tc_matvec(partials.reshape(_NW, EMBED), W.T, b)
